# baseline (speedup 1.0000x reference)
import functools

import jax
import jax.numpy as jnp
from jax import lax
from jax.experimental import pallas as pl
from jax.experimental.pallas import tpu as pltpu
from jax.experimental.pallas import tpu_sc as plsc

VOC = 1_000_000
DIM = 32
ROWS = 16384
COLS = 26
PAD = 128
NC = 2
NS = 16
NW = NC * NS
BPW = ROWS // NW
CHB = 256
NCHUNK = COLS * (BPW // CHB)
NBUF = 2


@functools.partial(
    pl.kernel,
    out_type=jax.ShapeDtypeStruct((COLS, DIM, ROWS), jnp.float32),
    mesh=plsc.VectorSubcoreMesh(core_axis_name="c", subcore_axis_name="s"),
    scratch_types=(
        [pltpu.VMEM((COLS * BPW,), jnp.int32)]
        + [pltpu.VMEM((CHB, PAD), jnp.float32) for _ in range(NBUF)]
        + [pltpu.VMEM((DIM, CHB), jnp.float32) for _ in range(NBUF)]
        + [pltpu.SemaphoreType.DMA for _ in range(NBUF)]
        + [pltpu.SemaphoreType.DMA]
    ),
    compiler_params=pltpu.CompilerParams(
        use_tc_tiling_on_sc=True, needs_layout_passes=False),
)
def _sc_gather(xt_hbm, tpad_hbm, out_hbm, idx_v, *bufs):
    wid = lax.axis_index("s") * NC + lax.axis_index("c")
    b0 = wid * BPW

    wide = bufs[:NBUF]
    comp = bufs[NBUF:2 * NBUF]
    sems = bufs[2 * NBUF:3 * NBUF]
    isem = bufs[3 * NBUF]

    for s in range(COLS):
        pltpu.async_copy(
            xt_hbm.at[s, pl.ds(b0, BPW)],
            idx_v.at[pl.ds(s * BPW, BPW)], isem)
    for s in range(COLS):
        pltpu.make_async_copy(
            xt_hbm.at[s, pl.ds(b0, BPW)],
            idx_v.at[pl.ds(s * BPW, BPW)], isem).wait()

    def start_gather(c, b):
        pltpu.async_copy(
            tpad_hbm.at[idx_v.at[pl.ds(c * CHB, CHB)]], wide[b], sems[b])

    def wait_gather(c, b):
        pltpu.make_async_copy(
            tpad_hbm.at[idx_v.at[pl.ds(c * CHB, CHB)]], wide[b],
            sems[b]).wait()

    iota = lax.iota(jnp.int32, 16)

    def extract(b):
        def grp(g, carry):
            kvec = g * 16 + iota
            for d in range(DIM):
                dvec = jnp.full((16,), d, jnp.int32)
                val = plsc.load_gather(wide[b], [kvec, dvec])
                plsc.store_scatter(comp[b], [dvec, kvec], val)
            return carry
        lax.fori_loop(0, CHB // 16, grp, 0)

    def writeback(c, b):
        s = c >> 1
        bb = b0 + (c & 1) * CHB
        pltpu.sync_copy(comp[b], out_hbm.at[s, :, pl.ds(bb, CHB)])

    def step(c, b):
        wait_gather(c, b)
        extract(b)
        writeback(c, b)

    for b in range(NBUF):
        start_gather(b, b)

    def body(i, carry):
        for b in range(NBUF):
            c = i * NBUF + b
            step(c, b)
            start_gather(c + NBUF, b)
        return carry

    lax.fori_loop(0, (NCHUNK - NBUF) // NBUF, body, 0)
    for c in range(NCHUNK - NBUF, NCHUNK):
        step(c, c % NBUF)


def kernel(x, table):
    tpad = jnp.pad(table, ((0, 0), (0, PAD - DIM)))
    out = _sc_gather(x.T, tpad)
    return out.transpose(2, 0, 1)
